# baseline (speedup 1.0000x reference)
import functools

import jax
import jax.numpy as jnp
from jax import lax
from jax.experimental import pallas as pl
from jax.experimental.pallas import tpu as pltpu
from jax.experimental.pallas import tpu_sc as plsc

_NUM_CLASSES = 1000000
_HIDDEN = 64
_BATCH = 16384

_L = 16
_NW = 32
_B_PER_W = _BATCH // _NW
_GCHUNK = 128
_NCHUNK = _B_PER_W // _GCHUNK


def _embed_kernel(labels_hbm, drop_hbm, table_hbm, out_hbm,
                  idx_v, lab_v, drop_v, rows_v, sem):
    wid = lax.axis_index("s") * 2 + lax.axis_index("c")
    base = wid * _B_PER_W

    cp_lab = pltpu.make_async_copy(labels_hbm.at[pl.ds(base, _B_PER_W)], lab_v, sem)
    cp_drop = pltpu.make_async_copy(drop_hbm.at[pl.ds(base, _B_PER_W)], drop_v, sem)
    cp_lab.start()
    cp_drop.start()
    cp_lab.wait()
    cp_drop.wait()

    for i in range(_B_PER_W // _L):
        sl = pl.ds(i * _L, _L)
        lab = lab_v[sl]
        drop = drop_v[sl]
        masked = jnp.where(drop == 1, jnp.full((_L,), _NUM_CLASSES, jnp.int32), lab)
        idx_v[i // (_GCHUNK // _L), pl.ds((i % (_GCHUNK // _L)) * _L, _L)] = masked

    copies = []
    for j in range(_NCHUNK):
        cp = pltpu.make_async_copy(
            table_hbm.at[idx_v.at[j]],
            rows_v.at[pl.ds(j * _GCHUNK, _GCHUNK)],
            sem,
        )
        cp.start()
        copies.append(cp)
    for cp in copies:
        cp.wait()

    pltpu.sync_copy(rows_v, out_hbm.at[pl.ds(base, _B_PER_W)])


@jax.jit
def _embed(labels, force_drop_ids, table):
    mesh = plsc.VectorSubcoreMesh(core_axis_name="c", subcore_axis_name="s")
    f = functools.partial(
        pl.kernel,
        mesh=mesh,
        out_type=jax.ShapeDtypeStruct((_BATCH, _HIDDEN), jnp.float32),
        scratch_types=[
            pltpu.VMEM((_NCHUNK, _GCHUNK), jnp.int32),
            pltpu.VMEM((_B_PER_W,), jnp.int32),
            pltpu.VMEM((_B_PER_W,), jnp.int32),
            pltpu.VMEM((_B_PER_W, _HIDDEN), jnp.float32),
            pltpu.SemaphoreType.DMA,
        ],
        compiler_params=pltpu.CompilerParams(use_tc_tiling_on_sc=False),
    )(_embed_kernel)
    return f(labels, force_drop_ids, table)


def kernel(labels, train, force_drop_ids, table):
    del train
    return _embed(labels.reshape(-1), force_drop_ids, table)


# speedup vs baseline: 1.1546x; 1.1546x over previous
import functools

import jax
import jax.numpy as jnp
from jax import lax
from jax.experimental import pallas as pl
from jax.experimental.pallas import tpu as pltpu
from jax.experimental.pallas import tpu_sc as plsc

_NUM_CLASSES = 1000000
_HIDDEN = 64
_BATCH = 16384

_NW = 32
_B_PER_W = _BATCH // _NW
_INFLIGHT = 16
_NITER = _B_PER_W // _INFLIGHT


def _embed_kernel(labels_hbm, drop_hbm, table_hbm, out_hbm,
                  lab_v, drop_v, idx_v, rows_v, sem):
    wid = lax.axis_index("s") * 2 + lax.axis_index("c")
    base = wid * _B_PER_W

    cp_lab = pltpu.make_async_copy(labels_hbm.at[pl.ds(base, _B_PER_W)], lab_v, sem)
    cp_drop = pltpu.make_async_copy(drop_hbm.at[pl.ds(base, _B_PER_W)], drop_v, sem)
    cp_lab.start()
    cp_drop.start()
    cp_lab.wait()
    cp_drop.wait()

    for i in range(_B_PER_W // 16):
        sl = pl.ds(i * 16, 16)
        idx_v[sl] = jnp.where(drop_v[sl] == 1,
                              jnp.full((16,), _NUM_CLASSES, jnp.int32),
                              lab_v[sl])

    @pl.loop(0, _NITER)
    def _gather(g):
        row0 = g * _INFLIGHT
        vec = idx_v[pl.ds(row0, 16)]
        copies = []
        for j in range(_INFLIGHT):
            cp = pltpu.make_async_copy(
                table_hbm.at[pl.ds(vec[j], 1), :],
                rows_v.at[pl.ds(row0 + j, 1), :],
                sem,
            )
            cp.start()
            copies.append(cp)
        for cp in copies:
            cp.wait()

    pltpu.sync_copy(rows_v, out_hbm.at[pl.ds(base, _B_PER_W)])


@jax.jit
def _embed(labels, force_drop_ids, table):
    mesh = plsc.VectorSubcoreMesh(core_axis_name="c", subcore_axis_name="s")
    f = functools.partial(
        pl.kernel,
        mesh=mesh,
        out_type=jax.ShapeDtypeStruct((_BATCH, _HIDDEN), jnp.float32),
        scratch_types=[
            pltpu.VMEM((_B_PER_W,), jnp.int32),
            pltpu.VMEM((_B_PER_W,), jnp.int32),
            pltpu.VMEM((_B_PER_W,), jnp.int32),
            pltpu.VMEM((_B_PER_W, _HIDDEN), jnp.float32),
            pltpu.SemaphoreType.DMA,
        ],
    )(_embed_kernel)
    return f(labels, force_drop_ids, table)


def kernel(labels, train, force_drop_ids, table):
    del train
    return _embed(labels.reshape(-1), force_drop_ids, table)


# speedup vs baseline: 1.3905x; 1.2043x over previous
import functools

import jax
import jax.numpy as jnp
from jax import lax
from jax.experimental import pallas as pl
from jax.experimental.pallas import tpu as pltpu
from jax.experimental.pallas import tpu_sc as plsc

_NUM_CLASSES = 1000000
_HIDDEN = 64
_BATCH = 16384

_NW = 32
_B_PER_W = _BATCH // _NW
_GRP = 16
_NITER = _B_PER_W // _GRP
_BOUNDS = (0, 250112, 500224, 750336, _NUM_CLASSES + 1)


def _make_slab_kernel(lo, hi, has_cfg):
    n_rows = hi - lo

    def body(labels_hbm, drop_hbm, slab_hbm, out_hbm,
             lab_v, drop_v, idx_v, cfg_v, rows_v, sem):
        wid = lax.axis_index("s") * 2 + lax.axis_index("c")
        base = wid * _B_PER_W

        cp_lab = pltpu.make_async_copy(
            labels_hbm.at[pl.ds(base, _B_PER_W)], lab_v, sem)
        cp_drop = pltpu.make_async_copy(
            drop_hbm.at[pl.ds(base, _B_PER_W)], drop_v, sem)
        cp_lab.start()
        cp_drop.start()
        if has_cfg:
            cp_cfg = pltpu.make_async_copy(
                slab_hbm.at[pl.ds(n_rows - 1, 1), :], cfg_v, sem)
            cp_cfg.start()
            cp_cfg.wait()
        cp_lab.wait()
        cp_drop.wait()

        for i in range(_B_PER_W // 16):
            sl = pl.ds(i * 16, 16)
            idx_v[sl] = jnp.where(drop_v[sl] == 1,
                                  jnp.full((16,), _NUM_CLASSES, jnp.int32),
                                  lab_v[sl])

        zero = jnp.zeros((16,), jnp.float32)
        @pl.loop(0, _B_PER_W // 4)
        def _zero(z):
            row = rows_v.at[z * 4 + 0]
            for k in range(4):
                rows_v.at[z * 4 + 0][pl.ds(k * 16, 16)] = zero
                rows_v.at[z * 4 + 1][pl.ds(k * 16, 16)] = zero
                rows_v.at[z * 4 + 2][pl.ds(k * 16, 16)] = zero
                rows_v.at[z * 4 + 3][pl.ds(k * 16, 16)] = zero

        if has_cfg:
            cfg_row = cfg_v.at[0]
            cfg_regs = [cfg_row[pl.ds(k * 16, 16)] for k in range(4)]
            @pl.loop(0, _NITER)
            def _fill(g):
                row0 = g * _GRP
                dvec = drop_v[pl.ds(row0, 16)]
                for j in range(_GRP):
                    p = row0 + j

                    @pl.when(dvec[j] == 1)
                    def _():
                        for k in range(4):
                            rows_v.at[p][pl.ds(k * 16, 16)] = cfg_regs[k]

        @pl.loop(0, _NITER)
        def _gather(g):
            row0 = g * _GRP
            vec = idx_v[pl.ds(row0, 16)]
            dvec = drop_v[pl.ds(row0, 16)] if has_cfg else None
            for phase in range(2):
                for j in range(_GRP):
                    sj = vec[j]
                    ok = (sj >= lo) & (sj < hi)
                    if has_cfg:
                        ok = ok & (dvec[j] == 0)
                    cp = pltpu.make_async_copy(
                        slab_hbm.at[pl.ds(sj - lo, 1), :],
                        rows_v.at[pl.ds(row0 + j, 1), :],
                        sem,
                    )

                    @pl.when(ok)
                    def _():
                        if phase == 0:
                            cp.start()
                        else:
                            cp.wait()

        pltpu.sync_copy(rows_v, out_hbm.at[pl.ds(base, _B_PER_W)])

    mesh = plsc.VectorSubcoreMesh(core_axis_name="c", subcore_axis_name="s")
    return functools.partial(
        pl.kernel,
        mesh=mesh,
        out_type=jax.ShapeDtypeStruct((_BATCH, _HIDDEN), jnp.float32),
        scratch_types=[
            pltpu.VMEM((_B_PER_W,), jnp.int32),
            pltpu.VMEM((_B_PER_W,), jnp.int32),
            pltpu.VMEM((_B_PER_W,), jnp.int32),
            pltpu.VMEM((1, _HIDDEN), jnp.float32),
            pltpu.VMEM((_B_PER_W, _HIDDEN), jnp.float32),
            pltpu.SemaphoreType.DMA,
        ],
    )(body)


@jax.jit
def _embed(labels, force_drop_ids, table):
    partials = []
    for k in range(4):
        lo, hi = _BOUNDS[k], _BOUNDS[k + 1]
        slab = lax.slice(table, (lo, 0), (hi, _HIDDEN))
        f = _make_slab_kernel(lo, hi, has_cfg=(k == 3))
        partials.append(f(labels, force_drop_ids, slab))
    return partials[0] + partials[1] + partials[2] + partials[3]


def kernel(labels, train, force_drop_ids, table):
    del train
    return _embed(labels.reshape(-1), force_drop_ids, table)


# speedup vs baseline: 2.0547x; 1.4777x over previous
import functools

import jax
import jax.numpy as jnp
from jax import lax
from jax.experimental import pallas as pl
from jax.experimental.pallas import tpu as pltpu
from jax.experimental.pallas import tpu_sc as plsc

_NUM_CLASSES = 1000000
_HIDDEN = 64
_BATCH = 16384

_NW = 32
_B_PER_W = _BATCH // _NW
_GRP = 16
_NITER = _B_PER_W // _GRP
_BOUNDS = (0, 250112, 500224, 750336, _NUM_CLASSES + 1)


def _make_slab_kernel(lo, hi, has_cfg):
    n_rows = hi - lo

    def body(labels_hbm, drop_hbm, slab_hbm, out_hbm,
             lab_v, drop_v, idx_v, cfg_v, rows_v, sem):
        wid = lax.axis_index("s") * 2 + lax.axis_index("c")
        base = wid * _B_PER_W

        cp_lab = pltpu.make_async_copy(
            labels_hbm.at[pl.ds(base, _B_PER_W)], lab_v, sem)
        cp_drop = pltpu.make_async_copy(
            drop_hbm.at[pl.ds(base, _B_PER_W)], drop_v, sem)
        cp_lab.start()
        cp_drop.start()
        if has_cfg:
            cp_cfg = pltpu.make_async_copy(
                slab_hbm.at[pl.ds(n_rows - 1, 1), :], cfg_v, sem)
            cp_cfg.start()
            cp_cfg.wait()
        cp_lab.wait()
        cp_drop.wait()

        for i in range(_B_PER_W // 16):
            sl = pl.ds(i * 16, 16)
            idx_v[sl] = jnp.where(drop_v[sl] == 1,
                                  jnp.full((16,), _NUM_CLASSES, jnp.int32),
                                  lab_v[sl])

        zero = jnp.zeros((16,), jnp.float32)
        @pl.loop(0, _B_PER_W // 4)
        def _zero(z):
            row = rows_v.at[z * 4 + 0]
            for k in range(4):
                rows_v.at[z * 4 + 0][pl.ds(k * 16, 16)] = zero
                rows_v.at[z * 4 + 1][pl.ds(k * 16, 16)] = zero
                rows_v.at[z * 4 + 2][pl.ds(k * 16, 16)] = zero
                rows_v.at[z * 4 + 3][pl.ds(k * 16, 16)] = zero

        if has_cfg:
            cfg_row = cfg_v.at[0]
            cfg_regs = [cfg_row[pl.ds(k * 16, 16)] for k in range(4)]
            @pl.loop(0, _NITER)
            def _fill(g):
                row0 = g * _GRP
                dvec = drop_v[pl.ds(row0, 16)]
                for j in range(_GRP):
                    p = row0 + j

                    @pl.when(dvec[j] == 1)
                    def _():
                        for k in range(4):
                            rows_v.at[p][pl.ds(k * 16, 16)] = cfg_regs[k]

        @pl.loop(0, _NITER)
        def _gather(g):
            row0 = g * _GRP
            vec = idx_v[pl.ds(row0, 16)]
            dvec = drop_v[pl.ds(row0, 16)] if has_cfg else None
            for phase in range(2):
                for j in range(_GRP):
                    sj = vec[j]
                    ok = (sj >= lo) & (sj < hi)
                    if has_cfg:
                        ok = ok & (dvec[j] == 0)
                    cp = pltpu.make_async_copy(
                        slab_hbm.at[pl.ds(sj - lo, 1), :],
                        rows_v.at[pl.ds(row0 + j, 1), :],
                        sem,
                    )

                    @pl.when(ok)
                    def _():
                        if phase == 0:
                            cp.start()
                        else:
                            cp.wait()

        pltpu.sync_copy(rows_v, out_hbm.at[pl.ds(base, _B_PER_W)])

    mesh = plsc.VectorSubcoreMesh(core_axis_name="c", subcore_axis_name="s")
    return functools.partial(
        pl.kernel,
        mesh=mesh,
        out_type=jax.ShapeDtypeStruct((_BATCH, _HIDDEN), jnp.float32),
        scratch_types=[
            pltpu.VMEM((_B_PER_W,), jnp.int32),
            pltpu.VMEM((_B_PER_W,), jnp.int32),
            pltpu.VMEM((_B_PER_W,), jnp.int32),
            pltpu.VMEM((1, _HIDDEN), jnp.float32),
            pltpu.VMEM((_B_PER_W, _HIDDEN), jnp.float32),
            pltpu.SemaphoreType.DMA,
        ],
    )(body)


@jax.jit
def _embed(labels, force_drop_ids, table):
    f = _make_slab_kernel(0, _NUM_CLASSES + 1, has_cfg=True)
    return f(labels, force_drop_ids, table)


def kernel(labels, train, force_drop_ids, table):
    del train
    return _embed(labels.reshape(-1), force_drop_ids, table)


# speedup vs baseline: 2.0587x; 1.0019x over previous
import functools

import jax
import jax.numpy as jnp
from jax import lax
from jax.experimental import pallas as pl
from jax.experimental.pallas import tpu as pltpu
from jax.experimental.pallas import tpu_sc as plsc

_NUM_CLASSES = 1000000
_HIDDEN = 64
_BATCH = 16384

_NW = 32
_B_PER_W = _BATCH // _NW
_GRP = 16
_NITER = _B_PER_W // _GRP


def _embed_body(labels_hbm, drop_hbm, table_hbm, out_hbm,
                lab_v, drop_v, idx_v, cfg_v, rows_v, sem):
    wid = lax.axis_index("s") * 2 + lax.axis_index("c")
    base = wid * _B_PER_W

    cp_lab = pltpu.make_async_copy(
        labels_hbm.at[pl.ds(base, _B_PER_W)], lab_v, sem)
    cp_drop = pltpu.make_async_copy(
        drop_hbm.at[pl.ds(base, _B_PER_W)], drop_v, sem)
    cp_lab.start()
    cp_drop.start()
    cp_cfg = pltpu.make_async_copy(
        table_hbm.at[pl.ds(_NUM_CLASSES, 1), :], cfg_v, sem)
    cp_cfg.start()
    cp_cfg.wait()
    cp_lab.wait()
    cp_drop.wait()

    for i in range(_B_PER_W // 16):
        sl = pl.ds(i * 16, 16)
        idx_v[sl] = jnp.where(drop_v[sl] == 1,
                              jnp.full((16,), _NUM_CLASSES, jnp.int32),
                              lab_v[sl])

    cfg_row = cfg_v.at[0]
    cfg_regs = [cfg_row[pl.ds(k * 16, 16)] for k in range(4)]

    @pl.loop(0, _NITER)
    def _fill(g):
        row0 = g * _GRP
        dvec = drop_v[pl.ds(row0, 16)]
        for j in range(_GRP):
            p = row0 + j

            @pl.when(dvec[j] == 1)
            def _():
                for k in range(4):
                    rows_v.at[p][pl.ds(k * 16, 16)] = cfg_regs[k]

    @pl.loop(0, _NITER)
    def _gather(g):
        row0 = g * _GRP
        vec = idx_v[pl.ds(row0, 16)]
        dvec = drop_v[pl.ds(row0, 16)]
        for phase in range(2):
            for j in range(_GRP):
                ok = dvec[j] == 0
                cp = pltpu.make_async_copy(
                    table_hbm.at[pl.ds(vec[j], 1), :],
                    rows_v.at[pl.ds(row0 + j, 1), :],
                    sem,
                )

                @pl.when(ok)
                def _():
                    if phase == 0:
                        cp.start()
                    else:
                        cp.wait()

    pltpu.sync_copy(rows_v, out_hbm.at[pl.ds(base, _B_PER_W)])


@jax.jit
def _embed(labels, force_drop_ids, table):
    mesh = plsc.VectorSubcoreMesh(core_axis_name="c", subcore_axis_name="s")
    f = functools.partial(
        pl.kernel,
        mesh=mesh,
        out_type=jax.ShapeDtypeStruct((_BATCH, _HIDDEN), jnp.float32),
        scratch_types=[
            pltpu.VMEM((_B_PER_W,), jnp.int32),
            pltpu.VMEM((_B_PER_W,), jnp.int32),
            pltpu.VMEM((_B_PER_W,), jnp.int32),
            pltpu.VMEM((1, _HIDDEN), jnp.float32),
            pltpu.VMEM((_B_PER_W, _HIDDEN), jnp.float32),
            pltpu.SemaphoreType.DMA,
        ],
    )(_embed_body)
    return f(labels, force_drop_ids, table)


def kernel(labels, train, force_drop_ids, table):
    del train
    return _embed(labels.reshape(-1), force_drop_ids, table)
